# baseline (speedup 1.0000x reference)
import functools

import jax
import jax.numpy as jnp
from jax import lax
from jax.experimental import pallas as pl
from jax.experimental.pallas import tpu as pltpu
from jax.experimental.pallas import tpu_sc as plsc

_LANES = 16
_SUB = 8


@functools.lru_cache(maxsize=None)
def _make_sc_kernel(B, D, NC, NS):
    NW = NC * NS
    bpw = B // NW
    groups = bpw // _LANES
    assert groups % 2 == 0
    mesh = plsc.VectorSubcoreMesh(core_axis_name="c", subcore_axis_name="s")
    buf_t = pltpu.VMEM((_LANES, _SUB, D), jnp.float32)

    @functools.partial(
        pl.kernel,
        mesh=mesh,
        compiler_params=pltpu.CompilerParams(
            needs_layout_passes=False, use_tc_tiling_on_sc=True),
        out_type=jax.ShapeDtypeStruct((B,), jnp.float32),
        scratch_types=[
            pltpu.VMEM((bpw,), jnp.int32),
            pltpu.VMEM((bpw,), jnp.int32),
            pltpu.VMEM((bpw,), jnp.int32),
            buf_t, buf_t, buf_t,
            buf_t, buf_t, buf_t,
            pltpu.VMEM((bpw,), jnp.float32),
            pltpu.SemaphoreType.DMA,
            pltpu.SemaphoreType.DMA,
        ],
    )
    def k(emb_hbm, rel_hbm, sidx_hbm, ridx_hbm, oidx_hbm, out_hbm,
          sidx_v, ridx_v, oidx_v, sbuf0, rbuf0, obuf0, sbuf1, rbuf1, obuf1,
          out_v, sem0, sem1):
        wid = lax.axis_index("s") * NC + lax.axis_index("c")
        base = wid * bpw
        pltpu.sync_copy(sidx_hbm.at[pl.ds(base, bpw)], sidx_v)
        pltpu.sync_copy(ridx_hbm.at[pl.ds(base, bpw)], ridx_v)
        pltpu.sync_copy(oidx_hbm.at[pl.ds(base, bpw)], oidx_v)

        bufs = ((sbuf0, rbuf0, obuf0, sem0), (sbuf1, rbuf1, obuf1, sem1))
        iota = lax.iota(jnp.int32, _LANES)
        n_chunks_d = D // _LANES

        def issue(g, slot):
            sb, rb, ob, sem = bufs[slot]
            gsl = pl.ds(g * _LANES, _LANES)
            stid = lax.shift_right_logical(sidx_v[gsl], 3)
            rtid = lax.shift_right_logical(ridx_v[gsl], 3)
            otid = lax.shift_right_logical(oidx_v[gsl], 3)
            for l in range(_LANES):
                pltpu.async_copy(emb_hbm.at[stid[l]], sb.at[l], sem)
                pltpu.async_copy(rel_hbm.at[rtid[l]], rb.at[l], sem)
                pltpu.async_copy(emb_hbm.at[otid[l]], ob.at[l], sem)

        def drain(slot):
            sb, rb, ob, sem = bufs[slot]
            for l in range(_LANES):
                pltpu.make_async_copy(emb_hbm.at[0], sb.at[l], sem).wait()
                pltpu.make_async_copy(rel_hbm.at[0], rb.at[l], sem).wait()
                pltpu.make_async_copy(emb_hbm.at[0], ob.at[l], sem).wait()

        def compute(g, slot):
            sb, rb, ob, _ = bufs[slot]
            gsl = pl.ds(g * _LANES, _LANES)
            ssub = jnp.bitwise_and(sidx_v[gsl], 7)
            rsub = jnp.bitwise_and(ridx_v[gsl], 7)
            osub = jnp.bitwise_and(oidx_v[gsl], 7)
            out_vec = jnp.zeros((_LANES,), jnp.float32)
            for l in range(_LANES):
                acc = None
                for c in range(n_chunks_d):
                    sl = pl.ds(c * _LANES, _LANES)
                    prod = (sb[l, ssub[l], sl] * rb[l, rsub[l], sl]
                            * ob[l, osub[l], sl])
                    acc = prod if acc is None else acc + prod
                out_vec = jnp.where(iota == l, jnp.sum(acc), out_vec)
            out_v[gsl] = out_vec

        issue(0, 0)

        def body(p, carry):
            g0 = p * 2
            issue(g0 + 1, 1)
            drain(0)
            compute(g0, 0)
            issue(g0 + 2, 0)
            drain(1)
            compute(g0 + 1, 1)
            return carry

        lax.fori_loop(0, groups // 2 - 1, body, 0)
        g0 = groups - 2
        issue(g0 + 1, 1)
        drain(0)
        compute(g0, 0)
        drain(1)
        compute(g0 + 1, 1)
        pltpu.sync_copy(out_v, out_hbm.at[pl.ds(base, bpw)])

    return k


def kernel(embeddings, relations, batch_subj_index, rel_index, batch_obj_index):
    B = batch_subj_index.shape[0]
    D = embeddings.shape[1]
    info = plsc.get_sparse_core_info()
    k = _make_sc_kernel(B, D, info.num_cores, info.num_subcores)
    emb3 = embeddings.reshape(embeddings.shape[0] // _SUB, _SUB, D)
    rel3 = relations.reshape(relations.shape[0] // _SUB, _SUB, D)
    return k(emb3, rel3,
             batch_subj_index.astype(jnp.int32),
             rel_index.astype(jnp.int32),
             batch_obj_index.astype(jnp.int32))


# speedup vs baseline: 1.2401x; 1.2401x over previous
import functools

import jax
import jax.numpy as jnp
from jax import lax
from jax.experimental import pallas as pl
from jax.experimental.pallas import tpu as pltpu
from jax.experimental.pallas import tpu_sc as plsc

_LANES = 16
_SUB = 8


@functools.lru_cache(maxsize=None)
def _make_sc_kernel(B, D, R, NC, NS):
    NW = NC * NS
    bpw = B // NW
    groups = bpw // _LANES
    assert groups % 2 == 0
    mesh = plsc.VectorSubcoreMesh(core_axis_name="c", subcore_axis_name="s")
    buf_t = pltpu.VMEM((_LANES, D), jnp.float32)

    @functools.partial(
        pl.kernel,
        mesh=mesh,
        compiler_params=pltpu.CompilerParams(
            needs_layout_passes=False, use_tc_tiling_on_sc=True),
        out_type=jax.ShapeDtypeStruct((B,), jnp.float32),
        scratch_types=[
            pltpu.VMEM((bpw,), jnp.int32),
            pltpu.VMEM((bpw,), jnp.int32),
            pltpu.VMEM((bpw,), jnp.int32),
            pltpu.VMEM((R // 2, 2 * D), jnp.float32),
            buf_t, buf_t,
            buf_t, buf_t,
            pltpu.VMEM((bpw,), jnp.float32),
            pltpu.SemaphoreType.DMA,
            pltpu.SemaphoreType.DMA,
        ],
    )
    def k(emb_hbm, rel_hbm, sidx_hbm, ridx_hbm, oidx_hbm, out_hbm,
          sidx_v, ridx_v, oidx_v, rel_v, sbuf0, obuf0, sbuf1, obuf1,
          out_v, sem0, sem1):
        wid = lax.axis_index("s") * NC + lax.axis_index("c")
        base = wid * bpw
        pltpu.sync_copy(sidx_hbm.at[pl.ds(base, bpw)], sidx_v)
        pltpu.sync_copy(ridx_hbm.at[pl.ds(base, bpw)], ridx_v)
        pltpu.sync_copy(oidx_hbm.at[pl.ds(base, bpw)], oidx_v)
        pltpu.sync_copy(rel_hbm, rel_v)

        bufs = ((sbuf0, obuf0, sem0), (sbuf1, obuf1, sem1))
        iota = lax.iota(jnp.int32, _LANES)
        n_chunks_d = D // _LANES

        def issue(g, slot):
            sb, ob, sem = bufs[slot]
            gsl = pl.ds(g * _LANES, _LANES)
            sidx = sidx_v[gsl]
            oidx = oidx_v[gsl]
            stid = lax.shift_right_logical(sidx, 3)
            otid = lax.shift_right_logical(oidx, 3)
            ssub = jnp.bitwise_and(sidx, 7)
            osub = jnp.bitwise_and(oidx, 7)
            for l in range(_LANES):
                pltpu.async_copy(emb_hbm.at[stid[l], ssub[l]], sb.at[l], sem)
                pltpu.async_copy(emb_hbm.at[otid[l], osub[l]], ob.at[l], sem)

        def drain(slot):
            sb, ob, sem = bufs[slot]
            pltpu.make_async_copy(emb_hbm.at[pl.ds(0, _LANES), 0], sb, sem).wait()
            pltpu.make_async_copy(emb_hbm.at[pl.ds(0, _LANES), 0], ob, sem).wait()

        def compute(g, slot):
            sb, ob, _ = bufs[slot]
            gsl = pl.ds(g * _LANES, _LANES)
            ridx = ridx_v[gsl]
            rrow = lax.shift_right_logical(ridx, 1)
            rcol = jnp.bitwise_and(ridx, 1) * D
            out_vec = jnp.zeros((_LANES,), jnp.float32)
            for l in range(_LANES):
                acc = None
                for c in range(n_chunks_d):
                    sl = pl.ds(c * _LANES, _LANES)
                    rsl = pl.ds(rcol[l] + c * _LANES, _LANES)
                    prod = sb[l, sl] * ob[l, sl] * rel_v[rrow[l], rsl]
                    acc = prod if acc is None else acc + prod
                out_vec = jnp.where(iota == l, jnp.sum(acc), out_vec)
            out_v[gsl] = out_vec

        issue(0, 0)

        def body(p, carry):
            g0 = p * 2
            issue(g0 + 1, 1)
            drain(0)
            compute(g0, 0)
            issue(g0 + 2, 0)
            drain(1)
            compute(g0 + 1, 1)
            return carry

        lax.fori_loop(0, groups // 2 - 1, body, 0)
        g0 = groups - 2
        issue(g0 + 1, 1)
        drain(0)
        compute(g0, 0)
        drain(1)
        compute(g0 + 1, 1)
        pltpu.sync_copy(out_v, out_hbm.at[pl.ds(base, bpw)])

    return k


def kernel(embeddings, relations, batch_subj_index, rel_index, batch_obj_index):
    B = batch_subj_index.shape[0]
    D = embeddings.shape[1]
    R = relations.shape[0]
    info = plsc.get_sparse_core_info()
    k = _make_sc_kernel(B, D, R, info.num_cores, info.num_subcores)
    emb3 = embeddings.reshape(embeddings.shape[0] // _SUB, _SUB, D)
    rel2 = relations.reshape(R // 2, 2 * D)
    return k(emb3, rel2,
             batch_subj_index.astype(jnp.int32),
             rel_index.astype(jnp.int32),
             batch_obj_index.astype(jnp.int32))
